# Initial kernel scaffold; baseline (speedup 1.0000x reference)
import jax
import jax.numpy as jnp
from jax.experimental import pallas as pl


def kernel(x, table):
    raise NotImplementedError("write your pallas kernel here")


# speedup vs baseline: 1.0286x; 1.0286x over previous
import functools
import math

import jax
import jax.numpy as jnp
from jax import lax
from jax.experimental import pallas as pl
from jax.experimental.pallas import tpu as pltpu
from jax.experimental.pallas import tpu_sc as plsc

_DIM = 128
_SCALE = math.sqrt(128.0)

_NC = 2
_NS = 16
_NW = _NC * _NS

_CHUNK = 128


def _make_lookup(n_rows: int):
    assert n_rows % (_NW * _CHUNK) == 0
    per_w = n_rows // _NW
    n_chunks = per_w // _CHUNK
    mesh = plsc.VectorSubcoreMesh(
        core_axis_name="c", subcore_axis_name="s",
        num_cores=_NC, num_subcores=_NS,
    )

    @functools.partial(
        pl.kernel,
        out_type=jax.ShapeDtypeStruct((n_rows, _DIM), jnp.float32),
        mesh=mesh,
        scratch_types=[
            pltpu.VMEM((n_chunks, _CHUNK), jnp.int32),
            pltpu.VMEM((_CHUNK, _DIM), jnp.float32),
            pltpu.SemaphoreType.DMA,
        ],
    )
    def lookup(x_hbm, table_hbm, out_hbm, idx_v, buf, sem):
        wid = lax.axis_index("s") * _NC + lax.axis_index("c")
        pltpu.sync_copy(x_hbm.at[pl.ds(wid * n_chunks, n_chunks)], idx_v)
        row0 = wid * per_w

        def do_chunk(j, carry):
            pltpu.async_copy(table_hbm.at[idx_v.at[j]], buf, sem).wait()

            def scale_row(r, c):
                for k in range(_DIM // 16):
                    sl = pl.ds(k * 16, 16)
                    buf[r, sl] = buf[r, sl] * _SCALE
                return c

            lax.fori_loop(0, _CHUNK, scale_row, 0, unroll=2)
            pltpu.sync_copy(buf, out_hbm.at[pl.ds(row0 + j * _CHUNK, _CHUNK)])
            return carry

        lax.fori_loop(0, n_chunks, do_chunk, 0)

    return lookup


def kernel(x, table):
    orig_shape = x.shape
    n = 1
    for d in orig_shape:
        n *= d
    xf = x.reshape(_NW * (n // (_NW * _CHUNK)), _CHUNK).astype(jnp.int32)
    out = _make_lookup(n)(xf, table)
    return out.reshape(*orig_shape, _DIM)
